# Initial kernel scaffold; baseline (speedup 1.0000x reference)
import functools

import jax
import jax.numpy as jnp
from jax import lax
from jax.experimental import pallas as pl
from jax.experimental.pallas import tpu as pltpu
from jax.experimental.pallas import tpu_sc as plsc

NUM_CORES = 2
NUM_SUBCORES = 16
NUM_WORKERS = NUM_CORES * NUM_SUBCORES
CHUNK = 128


@functools.partial(jax.jit, static_argnums=(2, 3))
def _embedding_lookup(idx2d, table, n_chunks_per_w, dim):
    total = idx2d.shape[0] * CHUNK
    mesh = plsc.VectorSubcoreMesh(core_axis_name="c", subcore_axis_name="s")

    @functools.partial(
        pl.kernel,
        mesh=mesh,
        out_type=jax.ShapeDtypeStruct((total, dim), jnp.float32),
        scratch_types=[
            pltpu.VMEM((n_chunks_per_w, CHUNK), jnp.int32),
            pltpu.VMEM((CHUNK, dim), jnp.float32),
            pltpu.SemaphoreType.DMA,
        ],
    )
    def emb_kernel(idx_hbm, table_hbm, out_hbm, idx_v, rows_v, gsem):
        wid = lax.axis_index("s") * NUM_CORES + lax.axis_index("c")
        chunk_base = wid * n_chunks_per_w
        row_base = chunk_base * CHUNK
        pltpu.sync_copy(idx_hbm.at[pl.ds(chunk_base, n_chunks_per_w)], idx_v)

        def body(j, _):
            pltpu.async_copy(table_hbm.at[idx_v.at[j]], rows_v, gsem).wait()
            start = pl.multiple_of(row_base + j * CHUNK, CHUNK)
            pltpu.sync_copy(rows_v, out_hbm.at[pl.ds(start, CHUNK)])
            return 0

        lax.fori_loop(0, n_chunks_per_w, body, 0)

    return emb_kernel(idx2d, table)


def kernel(token_ids, embedding_matrix):
    orig_shape = token_ids.shape
    dim = embedding_matrix.shape[1]
    flat = token_ids.reshape(-1).astype(jnp.int32)
    total = flat.shape[0]
    assert total % (NUM_WORKERS * CHUNK) == 0
    n_chunks_per_w = total // (NUM_WORKERS * CHUNK)
    idx2d = flat.reshape(total // CHUNK, CHUNK)
    out = _embedding_lookup(idx2d, embedding_matrix, n_chunks_per_w, dim)
    return out.reshape(*orig_shape, dim)


# speedup vs baseline: 1.6839x; 1.6839x over previous
import functools

import jax
import jax.numpy as jnp
from jax import lax
from jax.experimental import pallas as pl
from jax.experimental.pallas import tpu as pltpu
from jax.experimental.pallas import tpu_sc as plsc

NUM_CORES = 2
NUM_SUBCORES = 16
NUM_WORKERS = NUM_CORES * NUM_SUBCORES
CHUNK = 128


@functools.partial(jax.jit, static_argnums=(2, 3))
def _embedding_lookup(idx2d, table, n_chunks_per_w, dim):
    total = idx2d.shape[0] * CHUNK
    mesh = plsc.VectorSubcoreMesh(core_axis_name="c", subcore_axis_name="s")

    @functools.partial(
        pl.kernel,
        mesh=mesh,
        out_type=jax.ShapeDtypeStruct((total, dim), jnp.float32),
        scratch_types=[
            pltpu.VMEM((n_chunks_per_w, CHUNK), jnp.int32),
            pltpu.VMEM((CHUNK, dim), jnp.float32),
            pltpu.SemaphoreType.DMA,
        ],
        compiler_params=pltpu.CompilerParams(use_tc_tiling_on_sc=False),
    )
    def emb_kernel(idx_hbm, table_hbm, out_hbm, idx_v, rows_v, gsem):
        wid = lax.axis_index("s") * NUM_CORES + lax.axis_index("c")
        chunk_base = wid * n_chunks_per_w
        row_base = chunk_base * CHUNK
        pltpu.sync_copy(idx_hbm.at[pl.ds(chunk_base, n_chunks_per_w)], idx_v)

        def body(j, _):
            pltpu.async_copy(table_hbm.at[idx_v.at[j]], rows_v, gsem).wait()
            start = pl.multiple_of(row_base + j * CHUNK, CHUNK)
            pltpu.sync_copy(rows_v, out_hbm.at[pl.ds(start, CHUNK)])
            return 0

        lax.fori_loop(0, n_chunks_per_w, body, 0)

    return emb_kernel(idx2d, table)


def kernel(token_ids, embedding_matrix):
    orig_shape = token_ids.shape
    dim = embedding_matrix.shape[1]
    flat = token_ids.reshape(-1).astype(jnp.int32)
    total = flat.shape[0]
    assert total % (NUM_WORKERS * CHUNK) == 0
    n_chunks_per_w = total // (NUM_WORKERS * CHUNK)
    idx2d = flat.reshape(total // CHUNK, CHUNK)
    out = _embedding_lookup(idx2d, embedding_matrix, n_chunks_per_w, dim)
    return out.reshape(*orig_shape, dim)


# speedup vs baseline: 1.8736x; 1.1127x over previous
import functools

import jax
import jax.numpy as jnp
from jax import lax
from jax.experimental import pallas as pl
from jax.experimental.pallas import tpu as pltpu
from jax.experimental.pallas import tpu_sc as plsc

NUM_CORES = 2
NUM_SUBCORES = 16
NUM_WORKERS = NUM_CORES * NUM_SUBCORES
CHUNK = 128
K = 8
G = 4


@functools.partial(jax.jit, static_argnums=(2, 3))
def _embedding_lookup(idx2d, table, n_chunks_per_w, dim):
    total = idx2d.shape[0] * CHUNK
    n = n_chunks_per_w
    assert n % K == 0
    mesh = plsc.VectorSubcoreMesh(core_axis_name="c", subcore_axis_name="s")

    @functools.partial(
        pl.kernel,
        mesh=mesh,
        out_type=jax.ShapeDtypeStruct((total, dim), jnp.float32),
        scratch_types=[
            pltpu.VMEM((n, CHUNK), jnp.int32),
            pltpu.VMEM((K, CHUNK, dim), jnp.float32),
            pltpu.SemaphoreType.DMA((K,)),
            pltpu.SemaphoreType.DMA((K,)),
        ],
        compiler_params=pltpu.CompilerParams(use_tc_tiling_on_sc=False),
    )
    def emb_kernel(idx_hbm, table_hbm, out_hbm, idx_v, rows_v, gsem, wsem):
        wid = lax.axis_index("s") * NUM_CORES + lax.axis_index("c")
        chunk_base = wid * n
        row_base = chunk_base * CHUNK
        pltpu.sync_copy(idx_hbm.at[pl.ds(chunk_base, n)], idx_v)

        def fire_gather(j, b):
            pltpu.async_copy(table_hbm.at[idx_v.at[j]], rows_v.at[b], gsem.at[b])

        def wait_gather(j, b):
            pltpu.make_async_copy(
                table_hbm.at[idx_v.at[j]], rows_v.at[b], gsem.at[b]
            ).wait()

        def out_slice(j):
            start = pl.multiple_of(row_base + j * CHUNK, CHUNK)
            return out_hbm.at[pl.ds(start, CHUNK)]

        def fire_write(j, b):
            pltpu.async_copy(rows_v.at[b], out_slice(j), wsem.at[b])

        def wait_write(j, b):
            pltpu.make_async_copy(rows_v.at[b], out_slice(j), wsem.at[b]).wait()

        for b in range(G):
            fire_gather(b, b)

        def group(g, _):
            for b in range(K):
                j = g * K + b
                bn = (b + G) % K
                wait_gather(j, b)
                fire_write(j, b)

                @pl.when(jnp.logical_and(j + G < n, j + G - K >= 0))
                def _():
                    wait_write(j + G - K, bn)

                @pl.when(j + G < n)
                def _():
                    fire_gather(j + G, bn)

            return 0

        lax.fori_loop(0, n // K, group, 0)
        for b in range(K):
            wait_write(n - K + b, b)

    return emb_kernel(idx2d, table)


def kernel(token_ids, embedding_matrix):
    orig_shape = token_ids.shape
    dim = embedding_matrix.shape[1]
    flat = token_ids.reshape(-1).astype(jnp.int32)
    total = flat.shape[0]
    assert total % (NUM_WORKERS * CHUNK) == 0
    n_chunks_per_w = total // (NUM_WORKERS * CHUNK)
    idx2d = flat.reshape(total // CHUNK, CHUNK)
    out = _embedding_lookup(idx2d, embedding_matrix, n_chunks_per_w, dim)
    return out.reshape(*orig_shape, dim)


# speedup vs baseline: 1.8737x; 1.0001x over previous
import functools

import jax
import jax.numpy as jnp
from jax import lax
from jax.experimental import pallas as pl
from jax.experimental.pallas import tpu as pltpu
from jax.experimental.pallas import tpu_sc as plsc

NUM_CORES = 2
NUM_SUBCORES = 16
NUM_WORKERS = NUM_CORES * NUM_SUBCORES
TOK_PER_CHUNK = 2
K = 8
G = 4


@functools.partial(jax.jit, static_argnums=(2,))
def _embedding_lookup(idx2d, table, seq):
    n_tokens = idx2d.shape[0] * TOK_PER_CHUNK
    dim = table.shape[1]
    chunk_idx = TOK_PER_CHUNK * seq
    n = idx2d.shape[0] // NUM_WORKERS
    assert n % K == 0
    mesh = plsc.VectorSubcoreMesh(core_axis_name="c", subcore_axis_name="s")

    @functools.partial(
        pl.kernel,
        mesh=mesh,
        out_type=jax.ShapeDtypeStruct((n_tokens, seq, dim), jnp.float32),
        scratch_types=[
            pltpu.VMEM((n, chunk_idx), jnp.int32),
            pltpu.VMEM((K, chunk_idx, dim), jnp.float32),
            pltpu.SemaphoreType.DMA((K,)),
            pltpu.SemaphoreType.DMA((K,)),
        ],
        compiler_params=pltpu.CompilerParams(use_tc_tiling_on_sc=False),
    )
    def emb_kernel(idx_hbm, table_hbm, out_hbm, idx_v, rows_v, gsem, wsem):
        wid = lax.axis_index("s") * NUM_CORES + lax.axis_index("c")
        chunk_base = wid * n
        tok_base = chunk_base * TOK_PER_CHUNK
        pltpu.sync_copy(idx_hbm.at[pl.ds(chunk_base, n)], idx_v)

        def fire_gather(j, b):
            pltpu.async_copy(table_hbm.at[idx_v.at[j]], rows_v.at[b], gsem.at[b])

        def wait_gather(j, b):
            pltpu.make_async_copy(
                table_hbm.at[idx_v.at[j]], rows_v.at[b], gsem.at[b]
            ).wait()

        def write_parts(j, b):
            tok = tok_base + j * TOK_PER_CHUNK
            return [
                (rows_v.at[b, pl.ds(t * seq, seq)], out_hbm.at[tok + t])
                for t in range(TOK_PER_CHUNK)
            ]

        def fire_write(j, b):
            for src, dst in write_parts(j, b):
                pltpu.async_copy(src, dst, wsem.at[b])

        def wait_write(j, b):
            for src, dst in write_parts(j, b):
                pltpu.make_async_copy(src, dst, wsem.at[b]).wait()

        for b in range(G):
            fire_gather(b, b)

        def group(g, _):
            for b in range(K):
                j = g * K + b
                bn = (b + G) % K
                wait_gather(j, b)
                fire_write(j, b)

                @pl.when(jnp.logical_and(j + G < n, j + G - K >= 0))
                def _():
                    wait_write(j + G - K, bn)

                @pl.when(j + G < n)
                def _():
                    fire_gather(j + G, bn)

            return 0

        lax.fori_loop(0, n // K, group, 0)
        for b in range(K):
            wait_write(n - K + b, b)

    return emb_kernel(idx2d, table)


def kernel(token_ids, embedding_matrix):
    n_tokens, seq = token_ids.shape
    flat = token_ids.reshape(-1).astype(jnp.int32)
    assert n_tokens % (NUM_WORKERS * TOK_PER_CHUNK) == 0
    idx2d = flat.reshape(n_tokens // TOK_PER_CHUNK, TOK_PER_CHUNK * seq)
    return _embedding_lookup(idx2d, embedding_matrix, seq)
